# Initial kernel scaffold; baseline (speedup 1.0000x reference)
import jax
import jax.numpy as jnp
from jax.experimental import pallas as pl


def kernel(z, edge_index):
    raise NotImplementedError("write your pallas kernel here")


# speedup vs baseline: 2.0999x; 2.0999x over previous
import functools

import jax
import jax.numpy as jnp
from jax import lax
from jax.experimental import pallas as pl
from jax.experimental.pallas import tpu as pltpu
from jax.experimental.pallas import tpu_sc as plsc

L = 16
NW = 32
D = 256
DV = D // L


def _decoder_body(E, C, z_hbm, src_hbm, dst_hbm, out_hbm,
                  sidx_v, didx_v, srows_v, drows_v, pbuf_v, outv_v,
                  sem_s, sem_d):
    nchunk = E // C
    kmax = (nchunk + NW - 1) // NW
    wid = lax.axis_index("c") * 16 + lax.axis_index("s")

    def chunk_body(k, _):
        cid = wid + k * NW

        @pl.when(cid < nchunk)
        def _():
            base = cid * C
            pltpu.sync_copy(src_hbm.at[pl.ds(base, C)], sidx_v)
            pltpu.sync_copy(dst_hbm.at[pl.ds(base, C)], didx_v)
            cp_s = pltpu.async_copy(z_hbm.at[sidx_v], srows_v, sem_s)
            cp_d = pltpu.async_copy(z_hbm.at[didx_v], drows_v, sem_d)
            cp_s.wait()
            cp_d.wait()

            lanes = lax.broadcasted_iota(jnp.int32, (L,), 0)

            def group_body(g, _):
                for e in range(L):
                    row = g * L + e
                    acc = (srows_v[row, pl.ds(0, L)]
                           * drows_v[row, pl.ds(0, L)])
                    for i in range(1, DV):
                        acc = acc + (srows_v[row, pl.ds(i * L, L)]
                                     * drows_v[row, pl.ds(i * L, L)])
                    pbuf_v[pl.ds(e * L, L)] = acc
                    for h in (8, 4, 2, 1):
                        acc = acc + pbuf_v[pl.ds(e * L + h, L)]
                        pbuf_v[pl.ds(e * L, L)] = acc
                res = jnp.zeros((L,), jnp.float32)
                for e in range(L):
                    xe = pbuf_v[pl.ds(15 * e, L)]
                    res = jnp.where(lanes == e, xe, res)
                outv_v[pl.ds(g * L, L)] = 1.0 / (1.0 + jnp.exp(-res))
                return 0

            lax.fori_loop(0, C // L, group_body, 0)
            pltpu.sync_copy(outv_v, out_hbm.at[pl.ds(base, C)])

        return 0

    lax.fori_loop(0, kmax, chunk_body, 0)


def kernel(z, edge_index):
    E = edge_index.shape[1]
    C = 128
    src = edge_index[0]
    dst = edge_index[1]

    mesh = plsc.VectorSubcoreMesh(core_axis_name="c", subcore_axis_name="s")
    body = functools.partial(_decoder_body, E, C)
    f = pl.kernel(
        body,
        out_type=jax.ShapeDtypeStruct((E,), jnp.float32),
        mesh=mesh,
        scratch_types=[
            pltpu.VMEM((C,), jnp.int32),
            pltpu.VMEM((C,), jnp.int32),
            pltpu.VMEM((C, D), jnp.float32),
            pltpu.VMEM((C, D), jnp.float32),
            pltpu.VMEM((L * L + L,), jnp.float32),
            pltpu.VMEM((C,), jnp.float32),
            pltpu.SemaphoreType.DMA,
            pltpu.SemaphoreType.DMA,
        ],
    )
    return f(z, src, dst)


# speedup vs baseline: 3.1767x; 1.5128x over previous
import functools

import jax
import jax.numpy as jnp
from jax import lax
from jax.experimental import pallas as pl
from jax.experimental.pallas import tpu as pltpu
from jax.experimental.pallas import tpu_sc as plsc

L = 16
NW = 32
D = 256
DV = D // L
C = 64
PB = 32


def _decoder_body(E, z_hbm, src_hbm, dst_hbm, out_hbm,
                  sidx_v, didx_v, s0_v, d0_v, s1_v, d1_v, pbuf_v, outv_v,
                  ss0, sd0, ss1, sd1):
    nchunk = E // C
    bnk = nchunk // NW
    rem = nchunk - bnk * NW
    maxnk = bnk + (1 if rem else 0)
    wid = lax.axis_index("c") * 16 + lax.axis_index("s")
    nk = jnp.where(wid < rem, bnk + 1, bnk)
    start_chunk = wid * bnk + jnp.minimum(wid, rem)
    ebase = start_chunk * C

    pltpu.sync_copy(src_hbm.at[pl.ds(ebase, maxnk * C)], sidx_v)
    pltpu.sync_copy(dst_hbm.at[pl.ds(ebase, maxnk * C)], didx_v)

    def start(c, sbuf, dbuf, ssem, dsem):
        pltpu.async_copy(z_hbm.at[sidx_v.at[pl.ds(c * C, C)]], sbuf, ssem)
        pltpu.async_copy(z_hbm.at[didx_v.at[pl.ds(c * C, C)]], dbuf, dsem)

    def wait(c, sbuf, dbuf, ssem, dsem):
        pltpu.make_async_copy(
            z_hbm.at[sidx_v.at[pl.ds(c * C, C)]], sbuf, ssem).wait()
        pltpu.make_async_copy(
            z_hbm.at[didx_v.at[pl.ds(c * C, C)]], dbuf, dsem).wait()

    lanes = lax.broadcasted_iota(jnp.int32, (L,), 0)

    def compute(c, sbuf, dbuf):
        def group_body(g, _):
            for e in range(L):
                row = g * L + e
                acc = (sbuf[row, pl.ds(0, L)] * dbuf[row, pl.ds(0, L)])
                for i in range(1, DV):
                    acc = acc + (sbuf[row, pl.ds(i * L, L)]
                                 * dbuf[row, pl.ds(i * L, L)])
                pbuf_v[pl.ds(e * PB, L)] = acc
                for h in (8, 4, 2, 1):
                    acc = acc + pbuf_v[pl.ds(e * PB + h, L)]
                    pbuf_v[pl.ds(e * PB, L)] = acc
            res = jnp.zeros((L,), jnp.float32)
            for e in range(L):
                res = jnp.where(lanes == e, pbuf_v[pl.ds((PB - 1) * e, L)],
                                res)
            outv_v[pl.ds(g * L, L)] = 1.0 / (1.0 + jnp.exp(-res))
            return 0

        lax.fori_loop(0, C // L, group_body, 0)
        pltpu.sync_copy(outv_v, out_hbm.at[pl.ds(ebase + c * C, C)])

    start(0, s0_v, d0_v, ss0, sd0)

    def pipe_body(kk, _):
        c0 = 2 * kk
        c1 = c0 + 1
        c2 = c0 + 2

        @pl.when(c1 < nk)
        def _():
            start(c1, s1_v, d1_v, ss1, sd1)

        @pl.when(c0 < nk)
        def _():
            wait(c0, s0_v, d0_v, ss0, sd0)
            compute(c0, s0_v, d0_v)

        @pl.when(c2 < nk)
        def _():
            start(c2, s0_v, d0_v, ss0, sd0)

        @pl.when(c1 < nk)
        def _():
            wait(c1, s1_v, d1_v, ss1, sd1)
            compute(c1, s1_v, d1_v)

        return 0

    lax.fori_loop(0, (maxnk + 1) // 2, pipe_body, 0)


def kernel(z, edge_index):
    E = edge_index.shape[1]
    nchunk = E // C
    bnk = nchunk // NW
    maxnk = bnk + (1 if nchunk % NW else 0)
    pad = maxnk * C * NW - E + C
    src = jnp.pad(edge_index[0], (0, pad))
    dst = jnp.pad(edge_index[1], (0, pad))

    mesh = plsc.VectorSubcoreMesh(core_axis_name="c", subcore_axis_name="s")
    body = functools.partial(_decoder_body, E)
    f = pl.kernel(
        body,
        out_type=jax.ShapeDtypeStruct((E,), jnp.float32),
        mesh=mesh,
        scratch_types=[
            pltpu.VMEM((maxnk * C,), jnp.int32),
            pltpu.VMEM((maxnk * C,), jnp.int32),
            pltpu.VMEM((C, D), jnp.float32),
            pltpu.VMEM((C, D), jnp.float32),
            pltpu.VMEM((C, D), jnp.float32),
            pltpu.VMEM((C, D), jnp.float32),
            pltpu.VMEM((L * PB + L,), jnp.float32),
            pltpu.VMEM((C,), jnp.float32),
            pltpu.SemaphoreType.DMA,
            pltpu.SemaphoreType.DMA,
            pltpu.SemaphoreType.DMA,
            pltpu.SemaphoreType.DMA,
        ],
    )
    return f(z, src, dst)
